# Initial kernel scaffold; baseline (speedup 1.0000x reference)
import jax
import jax.numpy as jnp
from jax.experimental import pallas as pl


def kernel(x, edge_index, Wc1, bc1, Wc2, bc2, Wc3, bc3, W1, b1, W2, b2):
    raise NotImplementedError("write your pallas kernel here")


# speedup vs baseline: 5.5902x; 5.5902x over previous
import functools

import jax
import jax.numpy as jnp
from jax import lax
from jax.experimental import pallas as pl
from jax.experimental.pallas import tpu as pltpu
from jax.experimental.pallas import tpu_sc as plsc

_N = 50000
_NPAD = 50048
_E = 800000
_EPAD = 819200
_BLKW = 128
_NBLK = _EPAD // _BLKW
_FC = 16
_NC = 2
_NS = 16


def _mesh():
    return plsc.VectorSubcoreMesh(
        core_axis_name="c", subcore_axis_name="s",
        num_cores=_NC, num_subcores=_NS)


def _spmm_call(n_chunks, table, src2d, dst2d, zeros):
    rows_per_tile = _NPAD // _NS
    blk_per_tile = _NBLK // _NS
    kb = 8
    nsup = blk_per_tile // kb
    cpc = n_chunks // _NC

    def body(table_r, src_r, dst_r, zeros_r, out_r, src_v, dst_v, rows_v,
             accum, sem):
        c = lax.axis_index("c")
        s = lax.axis_index("s")
        row0 = s * rows_per_tile
        for ci in range(cpc):
            chunk = c * cpc + ci
            tchunk = table_r.at[pl.ds(chunk * _NPAD, _NPAD)]
            pltpu.sync_copy(zeros_r.at[pl.ds(row0, rows_per_tile)],
                            accum.at[pl.ds(row0, rows_per_tile)])
            plsc.subcore_barrier()

            def step(g, carry):
                base = s * blk_per_tile + g * kb
                pltpu.sync_copy(src_r.at[pl.ds(base, kb)], src_v)
                pltpu.sync_copy(dst_r.at[pl.ds(base, kb)], dst_v)
                cps = [pltpu.async_copy(tchunk.at[src_v.at[j]],
                                        rows_v.at[pl.ds(j * _BLKW, _BLKW)],
                                        sem)
                       for j in range(kb)]
                for cp in cps:
                    cp.wait()
                for j in range(kb):
                    pltpu.sync_copy(rows_v.at[pl.ds(j * _BLKW, _BLKW)],
                                    accum.at[dst_v.at[j]], add=True)
                return carry

            lax.fori_loop(0, nsup, step, 0)
            plsc.subcore_barrier()
            pltpu.sync_copy(accum.at[pl.ds(row0, rows_per_tile)],
                            out_r.at[pl.ds(chunk * _NPAD + row0,
                                           rows_per_tile)])
            plsc.subcore_barrier()

    fn = pl.kernel(
        body,
        out_type=jax.ShapeDtypeStruct((n_chunks * _NPAD, _FC), jnp.float32),
        mesh=_mesh(),
        scratch_types=[
            pltpu.VMEM((kb, _BLKW), jnp.int32),
            pltpu.VMEM((kb, _BLKW), jnp.int32),
            pltpu.VMEM((kb * _BLKW, _FC), jnp.float32),
            pltpu.VMEM_SHARED((_NPAD, _FC), jnp.float32),
            pltpu.SemaphoreType.DMA,
        ],
        compiler_params=pltpu.CompilerParams(use_tc_tiling_on_sc=False))
    return fn(table, src2d, dst2d, zeros)


def _deg_call(src2d, ones, zeros):
    rows_per_tile = _NPAD // _NS
    blk_per_w = _NBLK // (_NC * _NS)
    kb = 8
    nsup = blk_per_w // kb

    def body(src_r, ones_r, zeros_r, out_r, idx_v, ones_v, accum):
        c = lax.axis_index("c")
        s = lax.axis_index("s")
        w = s * _NC + c
        row0 = s * rows_per_tile
        pltpu.sync_copy(ones_r, ones_v)
        pltpu.sync_copy(zeros_r.at[pl.ds(row0, rows_per_tile)],
                        accum.at[pl.ds(row0, rows_per_tile)])
        plsc.subcore_barrier()

        def step(g, carry):
            base = w * blk_per_w + g * kb
            pltpu.sync_copy(src_r.at[pl.ds(base, kb)], idx_v)
            for j in range(kb):
                pltpu.sync_copy(ones_v, accum.at[idx_v.at[j]], add=True)
            return carry

        lax.fori_loop(0, nsup, step, 0)
        plsc.subcore_barrier()
        pltpu.sync_copy(accum.at[pl.ds(row0, rows_per_tile)],
                        out_r.at[pl.ds(c * _NPAD + row0, rows_per_tile)])

    fn = pl.kernel(
        body,
        out_type=jax.ShapeDtypeStruct((_NC * _NPAD, 16), jnp.float32),
        mesh=_mesh(),
        scratch_types=[
            pltpu.VMEM((kb, _BLKW), jnp.int32),
            pltpu.VMEM((_BLKW, 16), jnp.float32),
            pltpu.VMEM_SHARED((_NPAD, 16), jnp.float32),
        ],
        compiler_params=pltpu.CompilerParams(use_tc_tiling_on_sc=False))
    return fn(src2d, ones, zeros)


def _mm(A, W, b=None, C=None, relu=False, C2=None, b2=None):
    n, k = A.shape
    m = W.shape[1]
    bn = 400
    grid = (n // bn,)
    in_specs = [pl.BlockSpec((bn, k), lambda i: (i, 0)),
                pl.BlockSpec((k, m), lambda i: (0, 0))]
    args = [A, W]
    if b is not None:
        in_specs.append(pl.BlockSpec((1, m), lambda i: (0, 0)))
        args.append(b.reshape(1, m))
    if C is not None:
        in_specs.append(pl.BlockSpec((bn, m), lambda i: (i, 0)))
        args.append(C)
    if C2 is not None:
        in_specs.append(pl.BlockSpec((bn, m), lambda i: (i, 0)))
        args.append(C2)
    if b2 is not None:
        in_specs.append(pl.BlockSpec((1, m), lambda i: (0, 0)))
        args.append(b2.reshape(1, m))

    def body(*refs):
        i = 2
        acc = jnp.dot(refs[0][...], refs[1][...],
                      preferred_element_type=jnp.float32)
        if b is not None:
            acc = acc + refs[i][...]
            i += 1
        if C is not None:
            acc = acc + refs[i][...]
            i += 1
        if relu:
            acc = jnp.maximum(acc, 0.0)
        if C2 is not None:
            acc = acc + refs[i][...]
            i += 1
        if b2 is not None:
            acc = acc + refs[i][...]
            i += 1
        refs[i][...] = acc

    return pl.pallas_call(
        body, grid=grid, in_specs=in_specs,
        out_specs=pl.BlockSpec((bn, m), lambda i: (i, 0)),
        out_shape=jax.ShapeDtypeStruct((n, m), jnp.float32))(*args)


def _table(u, n_chunks):
    fp = n_chunks * _FC
    t = jnp.pad(u, ((0, _NPAD - _N), (0, fp - u.shape[1])))
    return t.reshape(_NPAD, n_chunks, _FC).transpose(1, 0, 2).reshape(
        n_chunks * _NPAD, _FC)


def _untable(o, n_chunks, f):
    return o.reshape(n_chunks, _NPAD, _FC).transpose(1, 0, 2).reshape(
        _NPAD, n_chunks * _FC)[:_N, :f]


def kernel(x, edge_index, Wc1, bc1, Wc2, bc2, Wc3, bc3, W1, b1, W2, b2):
    f32 = jnp.float32
    src = edge_index[0].astype(jnp.int32)
    dst = edge_index[1].astype(jnp.int32)
    padi = jnp.full((_EPAD - _E,), _N, jnp.int32)
    src2d = jnp.concatenate([src, padi]).reshape(_NBLK, _BLKW)
    dst2d = jnp.concatenate([dst, padi]).reshape(_NBLK, _BLKW)
    zerosfc = jnp.zeros((_NPAD, _FC), f32)
    zeros16 = jnp.zeros((_NPAD, 16), f32)
    ones16 = jnp.ones((_BLKW, 16), f32)

    degp = _deg_call(src2d, ones16, zeros16)
    deg = degp[:_N, 0] + degp[_NPAD:_NPAD + _N, 0]
    dinv = jnp.where(deg > 0, 1.0 / jnp.sqrt(jnp.maximum(deg, 1e-12)), 0.0)
    d = dinv[:, None]
    d2 = (dinv * dinv)[:, None]

    def S(u, n_chunks, f):
        t = _table(u, n_chunks)
        o = _spmm_call(n_chunks, t, src2d, dst2d, zerosfc)
        return _untable(o, n_chunks, f)

    S1 = S(d * x, 4, 64)
    S2 = S(-d2 * S1, 4, 64)
    Wbig = jnp.concatenate([Wc1[0] - Wc1[2], W1, W2], axis=1)
    X700 = _mm(x, Wbig)
    Wm2 = jnp.concatenate([-Wc1[1], -2.0 * Wc1[2]], axis=0)
    A12 = jnp.concatenate([d * S1, d * S2], axis=1)
    out1 = _mm(A12, Wm2, b=bc1, C=X700[:, :300], relu=True,
               C2=X700[:, 300:600], b2=b1)

    M3 = _mm(out1, jnp.concatenate([Wc2[0], Wc2[1], Wc2[2]], axis=1))
    A0 = M3[:, :100]
    A1 = M3[:, 100:200]
    A2 = M3[:, 200:300]
    SB = S(d * A2, 8, 100)
    SCr = S(d * A1 - 2.0 * d2 * SB, 8, 100)
    out3 = (jnp.maximum(A0 - A2 - d * SCr + bc2, 0.0)
            + jnp.maximum(X700[:, 600:700] + b2, 0.0))

    M4 = _mm(out3, jnp.concatenate([Wc3[0], Wc3[1], Wc3[2]], axis=1))
    B0 = M4[:, :64]
    B1 = M4[:, 64:128]
    B2 = M4[:, 128:192]
    SD = S(d * B2, 4, 64)
    SE = S(d * B1 - 2.0 * d2 * SD, 4, 64)
    return B0 - B2 - d * SE + bc3


# speedup vs baseline: 6.8962x; 1.2336x over previous
import functools

import jax
import jax.numpy as jnp
from jax import lax
from jax.experimental import pallas as pl
from jax.experimental.pallas import tpu as pltpu
from jax.experimental.pallas import tpu_sc as plsc

_N = 50000
_NPAD = 50048
_E = 800000
_EPAD = 819200
_BLKW = 128
_NBLK = _EPAD // _BLKW
_FC = 16
_NC = 2
_NS = 16


def _mesh():
    return plsc.VectorSubcoreMesh(
        core_axis_name="c", subcore_axis_name="s",
        num_cores=_NC, num_subcores=_NS)


def _spmm_call(n_chunks, table, src2d, dst2d, zeros):
    rows_per_tile = _NPAD // _NS
    blk_per_tile = _NBLK // _NS
    kb = 8
    nsup = blk_per_tile // kb
    cpc = n_chunks // _NC

    rb = kb * _BLKW

    def body(table_r, src_r, dst_r, zeros_r, out_r, src_v, dst_v, rows_v,
             accum, isem, dsem, gsem, ssem):
        c = lax.axis_index("c")
        s = lax.axis_index("s")
        row0 = s * rows_per_tile
        tile0 = s * blk_per_tile

        def fire_sidx(k, buf):
            kc = jnp.minimum(k, nsup - 1)
            return pltpu.async_copy(src_r.at[pl.ds(tile0 + kc * kb, kb)],
                                    src_v.at[pl.ds(buf * kb, kb)], isem)

        def fire_didx(k, buf):
            kc = jnp.minimum(k, nsup - 1)
            return pltpu.async_copy(dst_r.at[pl.ds(tile0 + kc * kb, kb)],
                                    dst_v.at[pl.ds(buf * kb, kb)], dsem)

        def wait_idx(idx_ref, sem):
            pltpu.make_async_copy(src_r.at[pl.ds(tile0, kb)],
                                  idx_ref.at[pl.ds(0, kb)], sem).wait()

        def wait_rows(sem):
            pltpu.make_async_copy(table_r.at[pl.ds(0, rb)],
                                  rows_v.at[pl.ds(0, rb)], sem).wait()

        for ci in range(cpc):
            chunk = c * cpc + ci
            tchunk = table_r.at[pl.ds(chunk * _NPAD, _NPAD)]
            pltpu.sync_copy(zeros_r.at[pl.ds(row0, rows_per_tile)],
                            accum.at[pl.ds(row0, rows_per_tile)])
            plsc.subcore_barrier()

            def fire_gathers(k, buf):
                for j in range(kb):
                    pltpu.async_copy(
                        tchunk.at[src_v.at[buf * kb + j]],
                        rows_v.at[pl.ds(buf * rb + j * _BLKW, _BLKW)], gsem)

            def fire_scatters(buf):
                for j in range(kb):
                    pltpu.async_copy(
                        rows_v.at[pl.ds(buf * rb + j * _BLKW, _BLKW)],
                        accum.at[dst_v.at[buf * kb + j]], ssem, add=True)

            fire_sidx(0, 0).wait()
            fire_didx(0, 0)
            fire_gathers(0, 0)
            fire_sidx(1, 1)

            def round_body(g, first):
                b = lax.rem(g, 2)
                bn = 1 - b
                if not first:
                    wait_rows(ssem)
                wait_idx(dst_v, dsem)
                fire_didx(g + 1, bn)
                wait_idx(src_v, isem)
                wait_rows(gsem)
                fire_gathers(g + 1, bn)
                fire_sidx(g + 2, b)
                fire_scatters(b)

            round_body(0, True)
            lax.fori_loop(1, nsup, lambda g, cr: (round_body(g, False), cr)[1],
                          0)
            wait_rows(ssem)
            wait_rows(gsem)
            wait_idx(src_v, isem)
            wait_idx(dst_v, dsem)
            plsc.subcore_barrier()
            pltpu.sync_copy(accum.at[pl.ds(row0, rows_per_tile)],
                            out_r.at[pl.ds(chunk * _NPAD + row0,
                                           rows_per_tile)])
            plsc.subcore_barrier()

    fn = pl.kernel(
        body,
        out_type=jax.ShapeDtypeStruct((n_chunks * _NPAD, _FC), jnp.float32),
        mesh=_mesh(),
        scratch_types=[
            pltpu.VMEM((2 * kb, _BLKW), jnp.int32),
            pltpu.VMEM((2 * kb, _BLKW), jnp.int32),
            pltpu.VMEM((2 * rb, _FC), jnp.float32),
            pltpu.VMEM_SHARED((_NPAD, _FC), jnp.float32),
            pltpu.SemaphoreType.DMA,
            pltpu.SemaphoreType.DMA,
            pltpu.SemaphoreType.DMA,
            pltpu.SemaphoreType.DMA,
        ],
        compiler_params=pltpu.CompilerParams(use_tc_tiling_on_sc=False))
    return fn(table, src2d, dst2d, zeros)


def _deg_call(src2d, ones, zeros):
    rows_per_tile = _NPAD // _NS
    blk_per_w = _NBLK // (_NC * _NS)
    kb = 8
    nsup = blk_per_w // kb

    def body(src_r, ones_r, zeros_r, out_r, idx_v, ones_v, accum):
        c = lax.axis_index("c")
        s = lax.axis_index("s")
        w = s * _NC + c
        row0 = s * rows_per_tile
        pltpu.sync_copy(ones_r, ones_v)
        pltpu.sync_copy(zeros_r.at[pl.ds(row0, rows_per_tile)],
                        accum.at[pl.ds(row0, rows_per_tile)])
        plsc.subcore_barrier()

        def step(g, carry):
            base = w * blk_per_w + g * kb
            pltpu.sync_copy(src_r.at[pl.ds(base, kb)], idx_v)
            for j in range(kb):
                pltpu.sync_copy(ones_v, accum.at[idx_v.at[j]], add=True)
            return carry

        lax.fori_loop(0, nsup, step, 0)
        plsc.subcore_barrier()
        pltpu.sync_copy(accum.at[pl.ds(row0, rows_per_tile)],
                        out_r.at[pl.ds(c * _NPAD + row0, rows_per_tile)])

    fn = pl.kernel(
        body,
        out_type=jax.ShapeDtypeStruct((_NC * _NPAD, 16), jnp.float32),
        mesh=_mesh(),
        scratch_types=[
            pltpu.VMEM((kb, _BLKW), jnp.int32),
            pltpu.VMEM((_BLKW, 16), jnp.float32),
            pltpu.VMEM_SHARED((_NPAD, 16), jnp.float32),
        ],
        compiler_params=pltpu.CompilerParams(use_tc_tiling_on_sc=False))
    return fn(src2d, ones, zeros)


def _mm(A, W, b=None, C=None, relu=False, C2=None, b2=None):
    n, k = A.shape
    m = W.shape[1]
    bn = 400
    grid = (n // bn,)
    in_specs = [pl.BlockSpec((bn, k), lambda i: (i, 0)),
                pl.BlockSpec((k, m), lambda i: (0, 0))]
    args = [A, W]
    if b is not None:
        in_specs.append(pl.BlockSpec((1, m), lambda i: (0, 0)))
        args.append(b.reshape(1, m))
    if C is not None:
        in_specs.append(pl.BlockSpec((bn, m), lambda i: (i, 0)))
        args.append(C)
    if C2 is not None:
        in_specs.append(pl.BlockSpec((bn, m), lambda i: (i, 0)))
        args.append(C2)
    if b2 is not None:
        in_specs.append(pl.BlockSpec((1, m), lambda i: (0, 0)))
        args.append(b2.reshape(1, m))

    def body(*refs):
        i = 2
        acc = jnp.dot(refs[0][...], refs[1][...],
                      preferred_element_type=jnp.float32)
        if b is not None:
            acc = acc + refs[i][...]
            i += 1
        if C is not None:
            acc = acc + refs[i][...]
            i += 1
        if relu:
            acc = jnp.maximum(acc, 0.0)
        if C2 is not None:
            acc = acc + refs[i][...]
            i += 1
        if b2 is not None:
            acc = acc + refs[i][...]
            i += 1
        refs[i][...] = acc

    return pl.pallas_call(
        body, grid=grid, in_specs=in_specs,
        out_specs=pl.BlockSpec((bn, m), lambda i: (i, 0)),
        out_shape=jax.ShapeDtypeStruct((n, m), jnp.float32))(*args)


def _table(u, n_chunks):
    fp = n_chunks * _FC
    t = jnp.pad(u, ((0, _NPAD - _N), (0, fp - u.shape[1])))
    return t.reshape(_NPAD, n_chunks, _FC).transpose(1, 0, 2).reshape(
        n_chunks * _NPAD, _FC)


def _untable(o, n_chunks, f):
    return o.reshape(n_chunks, _NPAD, _FC).transpose(1, 0, 2).reshape(
        _NPAD, n_chunks * _FC)[:_N, :f]


def kernel(x, edge_index, Wc1, bc1, Wc2, bc2, Wc3, bc3, W1, b1, W2, b2):
    f32 = jnp.float32
    src = edge_index[0].astype(jnp.int32)
    dst = edge_index[1].astype(jnp.int32)
    padi = jnp.full((_EPAD - _E,), _N, jnp.int32)
    src2d = jnp.concatenate([src, padi]).reshape(_NBLK, _BLKW)
    dst2d = jnp.concatenate([dst, padi]).reshape(_NBLK, _BLKW)
    zerosfc = jnp.zeros((_NPAD, _FC), f32)
    zeros16 = jnp.zeros((_NPAD, 16), f32)
    ones16 = jnp.ones((_BLKW, 16), f32)

    degp = _deg_call(src2d, ones16, zeros16)
    deg = degp[:_N, 0] + degp[_NPAD:_NPAD + _N, 0]
    dinv = jnp.where(deg > 0, 1.0 / jnp.sqrt(jnp.maximum(deg, 1e-12)), 0.0)
    d = dinv[:, None]
    d2 = (dinv * dinv)[:, None]

    def S(u, n_chunks, f):
        t = _table(u, n_chunks)
        o = _spmm_call(n_chunks, t, src2d, dst2d, zerosfc)
        return _untable(o, n_chunks, f)

    S1 = S(d * x, 4, 64)
    S2 = S(-d2 * S1, 4, 64)
    Wbig = jnp.concatenate([Wc1[0] - Wc1[2], W1, W2], axis=1)
    X700 = _mm(x, Wbig)
    Wm2 = jnp.concatenate([-Wc1[1], -2.0 * Wc1[2]], axis=0)
    A12 = jnp.concatenate([d * S1, d * S2], axis=1)
    out1 = _mm(A12, Wm2, b=bc1, C=X700[:, :300], relu=True,
               C2=X700[:, 300:600], b2=b1)

    M3 = _mm(out1, jnp.concatenate([Wc2[0], Wc2[1], Wc2[2]], axis=1))
    A0 = M3[:, :100]
    A1 = M3[:, 100:200]
    A2 = M3[:, 200:300]
    SB = S(d * A2, 8, 100)
    SCr = S(d * A1 - 2.0 * d2 * SB, 8, 100)
    out3 = (jnp.maximum(A0 - A2 - d * SCr + bc2, 0.0)
            + jnp.maximum(X700[:, 600:700] + b2, 0.0))

    M4 = _mm(out3, jnp.concatenate([Wc3[0], Wc3[1], Wc3[2]], axis=1))
    B0 = M4[:, :64]
    B1 = M4[:, 64:128]
    B2 = M4[:, 128:192]
    SD = S(d * B2, 4, 64)
    SE = S(d * B1 - 2.0 * d2 * SD, 4, 64)
    return B0 - B2 - d * SE + bc3


# speedup vs baseline: 6.9389x; 1.0062x over previous
import functools

import jax
import jax.numpy as jnp
from jax import lax
from jax.experimental import pallas as pl
from jax.experimental.pallas import tpu as pltpu
from jax.experimental.pallas import tpu_sc as plsc

_N = 50000
_NPAD = 50048
_E = 800000
_EPAD = 819200
_BLKW = 128
_NBLK = _EPAD // _BLKW
_FC = 16
_NC = 2
_NS = 16


def _mesh():
    return plsc.VectorSubcoreMesh(
        core_axis_name="c", subcore_axis_name="s",
        num_cores=_NC, num_subcores=_NS)


def _spmm_call(n_chunks, table, src2d, dst2d, zeros):
    rows_per_tile = _NPAD // _NS
    blk_per_tile = _NBLK // _NS
    kb = 16
    nsup = blk_per_tile // kb
    cpc = n_chunks // _NC

    rb = kb * _BLKW

    def body(table_r, src_r, dst_r, zeros_r, out_r, src_v, dst_v, rows_v,
             accum, isem, dsem, gsem, ssem):
        c = lax.axis_index("c")
        s = lax.axis_index("s")
        row0 = s * rows_per_tile
        tile0 = s * blk_per_tile

        def fire_sidx(k, buf):
            kc = jnp.minimum(k, nsup - 1)
            return pltpu.async_copy(src_r.at[pl.ds(tile0 + kc * kb, kb)],
                                    src_v.at[pl.ds(buf * kb, kb)], isem)

        def fire_didx(k, buf):
            kc = jnp.minimum(k, nsup - 1)
            return pltpu.async_copy(dst_r.at[pl.ds(tile0 + kc * kb, kb)],
                                    dst_v.at[pl.ds(buf * kb, kb)], dsem)

        def wait_idx(idx_ref, sem):
            pltpu.make_async_copy(src_r.at[pl.ds(tile0, kb)],
                                  idx_ref.at[pl.ds(0, kb)], sem).wait()

        def wait_rows(sem):
            pltpu.make_async_copy(table_r.at[pl.ds(0, rb)],
                                  rows_v.at[pl.ds(0, rb)], sem).wait()

        for ci in range(cpc):
            chunk = c * cpc + ci
            tchunk = table_r.at[pl.ds(chunk * _NPAD, _NPAD)]
            pltpu.sync_copy(zeros_r.at[pl.ds(row0, rows_per_tile)],
                            accum.at[pl.ds(row0, rows_per_tile)])
            plsc.subcore_barrier()

            def fire_gathers(k, buf):
                for j in range(kb):
                    pltpu.async_copy(
                        tchunk.at[src_v.at[buf * kb + j]],
                        rows_v.at[pl.ds(buf * rb + j * _BLKW, _BLKW)], gsem)

            def fire_scatters(buf):
                for j in range(kb):
                    pltpu.async_copy(
                        rows_v.at[pl.ds(buf * rb + j * _BLKW, _BLKW)],
                        accum.at[dst_v.at[buf * kb + j]], ssem, add=True)

            fire_sidx(0, 0).wait()
            fire_didx(0, 0)
            fire_gathers(0, 0)
            fire_sidx(1, 1)

            def round_body(g, first):
                b = lax.rem(g, 2)
                bn = 1 - b
                if not first:
                    wait_rows(ssem)
                wait_idx(dst_v, dsem)
                fire_didx(g + 1, bn)
                wait_idx(src_v, isem)
                wait_rows(gsem)
                fire_gathers(g + 1, bn)
                fire_sidx(g + 2, b)
                fire_scatters(b)

            round_body(0, True)
            lax.fori_loop(1, nsup, lambda g, cr: (round_body(g, False), cr)[1],
                          0)
            wait_rows(ssem)
            wait_rows(gsem)
            wait_idx(src_v, isem)
            wait_idx(dst_v, dsem)
            plsc.subcore_barrier()
            pltpu.sync_copy(accum.at[pl.ds(row0, rows_per_tile)],
                            out_r.at[pl.ds(chunk * _NPAD + row0,
                                           rows_per_tile)])
            plsc.subcore_barrier()

    fn = pl.kernel(
        body,
        out_type=jax.ShapeDtypeStruct((n_chunks * _NPAD, _FC), jnp.float32),
        mesh=_mesh(),
        scratch_types=[
            pltpu.VMEM((2 * kb, _BLKW), jnp.int32),
            pltpu.VMEM((2 * kb, _BLKW), jnp.int32),
            pltpu.VMEM((2 * rb, _FC), jnp.float32),
            pltpu.VMEM_SHARED((_NPAD, _FC), jnp.float32),
            pltpu.SemaphoreType.DMA,
            pltpu.SemaphoreType.DMA,
            pltpu.SemaphoreType.DMA,
            pltpu.SemaphoreType.DMA,
        ],
        compiler_params=pltpu.CompilerParams(use_tc_tiling_on_sc=False))
    return fn(table, src2d, dst2d, zeros)


def _deg_call(src2d, ones, zeros):
    rows_per_tile = _NPAD // _NS
    blk_per_w = _NBLK // (_NC * _NS)
    kb = 8
    nsup = blk_per_w // kb

    def body(src_r, ones_r, zeros_r, out_r, idx_v, ones_v, accum):
        c = lax.axis_index("c")
        s = lax.axis_index("s")
        w = s * _NC + c
        row0 = s * rows_per_tile
        pltpu.sync_copy(ones_r, ones_v)
        pltpu.sync_copy(zeros_r.at[pl.ds(row0, rows_per_tile)],
                        accum.at[pl.ds(row0, rows_per_tile)])
        plsc.subcore_barrier()

        def step(g, carry):
            base = w * blk_per_w + g * kb
            pltpu.sync_copy(src_r.at[pl.ds(base, kb)], idx_v)
            for j in range(kb):
                pltpu.sync_copy(ones_v, accum.at[idx_v.at[j]], add=True)
            return carry

        lax.fori_loop(0, nsup, step, 0)
        plsc.subcore_barrier()
        pltpu.sync_copy(accum.at[pl.ds(row0, rows_per_tile)],
                        out_r.at[pl.ds(c * _NPAD + row0, rows_per_tile)])

    fn = pl.kernel(
        body,
        out_type=jax.ShapeDtypeStruct((_NC * _NPAD, 16), jnp.float32),
        mesh=_mesh(),
        scratch_types=[
            pltpu.VMEM((kb, _BLKW), jnp.int32),
            pltpu.VMEM((_BLKW, 16), jnp.float32),
            pltpu.VMEM_SHARED((_NPAD, 16), jnp.float32),
        ],
        compiler_params=pltpu.CompilerParams(use_tc_tiling_on_sc=False))
    return fn(src2d, ones, zeros)


def _mm(A, W, b=None, C=None, relu=False, C2=None, b2=None):
    n, k = A.shape
    m = W.shape[1]
    bn = 400
    grid = (n // bn,)
    in_specs = [pl.BlockSpec((bn, k), lambda i: (i, 0)),
                pl.BlockSpec((k, m), lambda i: (0, 0))]
    args = [A, W]
    if b is not None:
        in_specs.append(pl.BlockSpec((1, m), lambda i: (0, 0)))
        args.append(b.reshape(1, m))
    if C is not None:
        in_specs.append(pl.BlockSpec((bn, m), lambda i: (i, 0)))
        args.append(C)
    if C2 is not None:
        in_specs.append(pl.BlockSpec((bn, m), lambda i: (i, 0)))
        args.append(C2)
    if b2 is not None:
        in_specs.append(pl.BlockSpec((1, m), lambda i: (0, 0)))
        args.append(b2.reshape(1, m))

    def body(*refs):
        i = 2
        acc = jnp.dot(refs[0][...], refs[1][...],
                      preferred_element_type=jnp.float32)
        if b is not None:
            acc = acc + refs[i][...]
            i += 1
        if C is not None:
            acc = acc + refs[i][...]
            i += 1
        if relu:
            acc = jnp.maximum(acc, 0.0)
        if C2 is not None:
            acc = acc + refs[i][...]
            i += 1
        if b2 is not None:
            acc = acc + refs[i][...]
            i += 1
        refs[i][...] = acc

    return pl.pallas_call(
        body, grid=grid, in_specs=in_specs,
        out_specs=pl.BlockSpec((bn, m), lambda i: (i, 0)),
        out_shape=jax.ShapeDtypeStruct((n, m), jnp.float32))(*args)


def _table(u, n_chunks):
    fp = n_chunks * _FC
    t = jnp.pad(u, ((0, _NPAD - _N), (0, fp - u.shape[1])))
    return t.reshape(_NPAD, n_chunks, _FC).transpose(1, 0, 2).reshape(
        n_chunks * _NPAD, _FC)


def _untable(o, n_chunks, f):
    return o.reshape(n_chunks, _NPAD, _FC).transpose(1, 0, 2).reshape(
        _NPAD, n_chunks * _FC)[:_N, :f]


def kernel(x, edge_index, Wc1, bc1, Wc2, bc2, Wc3, bc3, W1, b1, W2, b2):
    f32 = jnp.float32
    src = edge_index[0].astype(jnp.int32)
    dst = edge_index[1].astype(jnp.int32)
    padi = jnp.full((_EPAD - _E,), _N, jnp.int32)
    src2d = jnp.concatenate([src, padi]).reshape(_NBLK, _BLKW)
    dst2d = jnp.concatenate([dst, padi]).reshape(_NBLK, _BLKW)
    zerosfc = jnp.zeros((_NPAD, _FC), f32)
    zeros16 = jnp.zeros((_NPAD, 16), f32)
    ones16 = jnp.ones((_BLKW, 16), f32)

    degp = _deg_call(src2d, ones16, zeros16)
    deg = degp[:_N, 0] + degp[_NPAD:_NPAD + _N, 0]
    dinv = jnp.where(deg > 0, 1.0 / jnp.sqrt(jnp.maximum(deg, 1e-12)), 0.0)
    d = dinv[:, None]
    d2 = (dinv * dinv)[:, None]

    def S(u, n_chunks, f):
        t = _table(u, n_chunks)
        o = _spmm_call(n_chunks, t, src2d, dst2d, zerosfc)
        return _untable(o, n_chunks, f)

    S1 = S(d * x, 4, 64)
    S2 = S(-d2 * S1, 4, 64)
    Wbig = jnp.concatenate([Wc1[0] - Wc1[2], W1, W2], axis=1)
    X700 = _mm(x, Wbig)
    Wm2 = jnp.concatenate([-Wc1[1], -2.0 * Wc1[2]], axis=0)
    A12 = jnp.concatenate([d * S1, d * S2], axis=1)
    out1 = _mm(A12, Wm2, b=bc1, C=X700[:, :300], relu=True,
               C2=X700[:, 300:600], b2=b1)

    M3 = _mm(out1, jnp.concatenate([Wc2[0], Wc2[1], Wc2[2]], axis=1))
    A0 = M3[:, :100]
    A1 = M3[:, 100:200]
    A2 = M3[:, 200:300]
    SB = S(d * A2, 8, 100)
    SCr = S(d * A1 - 2.0 * d2 * SB, 8, 100)
    out3 = (jnp.maximum(A0 - A2 - d * SCr + bc2, 0.0)
            + jnp.maximum(X700[:, 600:700] + b2, 0.0))

    M4 = _mm(out3, jnp.concatenate([Wc3[0], Wc3[1], Wc3[2]], axis=1))
    B0 = M4[:, :64]
    B1 = M4[:, 64:128]
    B2 = M4[:, 128:192]
    SD = S(d * B2, 4, 64)
    SE = S(d * B1 - 2.0 * d2 * SD, 4, 64)
    return B0 - B2 - d * SE + bc3


# speedup vs baseline: 7.0299x; 1.0131x over previous
import functools

import jax
import jax.numpy as jnp
from jax import lax
from jax.experimental import pallas as pl
from jax.experimental.pallas import tpu as pltpu
from jax.experimental.pallas import tpu_sc as plsc

_N = 50000
_NPAD = 50048
_E = 800000
_EPAD = 819200
_BLKW = 128
_NBLK = _EPAD // _BLKW
_FC = 16
_NC = 2
_NS = 16


def _mesh():
    return plsc.VectorSubcoreMesh(
        core_axis_name="c", subcore_axis_name="s",
        num_cores=_NC, num_subcores=_NS)


def _spmm_call(n_chunks, table, src2d, dst2d, zeros):
    rows_per_tile = _NPAD // _NS
    blk_per_tile = _NBLK // _NS
    kb = 16
    nsup = blk_per_tile // kb
    cpc = n_chunks // _NC

    rb = kb * _BLKW

    def body(table_r, src_r, dst_r, zeros_r, out_r, src_v, dst_v, rows_v,
             accum, isem, dsem, gsem, ssem):
        c = lax.axis_index("c")
        s = lax.axis_index("s")
        row0 = s * rows_per_tile
        tile0 = s * blk_per_tile

        def fire_sidx(k, buf):
            kc = jnp.minimum(k, nsup - 1)
            return pltpu.async_copy(src_r.at[pl.ds(tile0 + kc * kb, kb)],
                                    src_v.at[pl.ds(buf * kb, kb)], isem)

        def fire_didx(k, buf):
            kc = jnp.minimum(k, nsup - 1)
            return pltpu.async_copy(dst_r.at[pl.ds(tile0 + kc * kb, kb)],
                                    dst_v.at[pl.ds(buf * kb, kb)], dsem)

        def wait_idx(idx_ref, sem):
            pltpu.make_async_copy(src_r.at[pl.ds(tile0, kb)],
                                  idx_ref.at[pl.ds(0, kb)], sem).wait()

        def wait_rows(sem):
            pltpu.make_async_copy(table_r.at[pl.ds(0, rb)],
                                  rows_v.at[pl.ds(0, rb)], sem).wait()

        for ci in range(cpc):
            chunk = c * cpc + ci
            tchunk = table_r.at[pl.ds(chunk * _NPAD, _NPAD)]
            pltpu.sync_copy(zeros_r.at[pl.ds(row0, rows_per_tile)],
                            accum.at[pl.ds(row0, rows_per_tile)])
            plsc.subcore_barrier()

            def fire_gathers(k, buf):
                for j in range(kb):
                    pltpu.async_copy(
                        tchunk.at[src_v.at[buf * kb + j]],
                        rows_v.at[pl.ds(buf * rb + j * _BLKW, _BLKW)], gsem)

            def fire_scatters(buf):
                for j in range(kb):
                    pltpu.async_copy(
                        rows_v.at[pl.ds(buf * rb + j * _BLKW, _BLKW)],
                        accum.at[dst_v.at[buf * kb + j]], ssem, add=True)

            fire_sidx(0, 0).wait()
            fire_didx(0, 0)
            fire_gathers(0, 0)
            fire_sidx(1, 1)

            def round_body(g, first):
                b = lax.rem(g, 2)
                bn = 1 - b
                if not first:
                    wait_rows(ssem)
                wait_idx(dst_v, dsem)
                fire_didx(g + 1, bn)
                wait_idx(src_v, isem)
                wait_rows(gsem)
                fire_gathers(g + 1, bn)
                fire_sidx(g + 2, b)
                fire_scatters(b)

            round_body(0, True)
            lax.fori_loop(1, nsup, lambda g, cr: (round_body(g, False), cr)[1],
                          0)
            wait_rows(ssem)
            wait_rows(gsem)
            wait_idx(src_v, isem)
            wait_idx(dst_v, dsem)
            plsc.subcore_barrier()
            pltpu.sync_copy(accum.at[pl.ds(row0, rows_per_tile)],
                            out_r.at[pl.ds(chunk * _NPAD + row0,
                                           rows_per_tile)])
            plsc.subcore_barrier()

    fn = pl.kernel(
        body,
        out_type=jax.ShapeDtypeStruct((n_chunks * _NPAD, _FC), jnp.float32),
        mesh=_mesh(),
        scratch_types=[
            pltpu.VMEM((2 * kb, _BLKW), jnp.int32),
            pltpu.VMEM((2 * kb, _BLKW), jnp.int32),
            pltpu.VMEM((2 * rb, _FC), jnp.float32),
            pltpu.VMEM_SHARED((_NPAD, _FC), jnp.float32),
            pltpu.SemaphoreType.DMA,
            pltpu.SemaphoreType.DMA,
            pltpu.SemaphoreType.DMA,
            pltpu.SemaphoreType.DMA,
        ],
        compiler_params=pltpu.CompilerParams(use_tc_tiling_on_sc=False))
    return fn(table, src2d, dst2d, zeros)


def _deg_call(src2d, ones, zeros):
    rows_per_tile = _NPAD // _NS
    blk_per_w = _NBLK // (_NC * _NS)
    kb = 8
    nsup = blk_per_w // kb

    def body(src_r, ones_r, zeros_r, out_r, idx_v, ones_v, accum, isem, ssem):
        c = lax.axis_index("c")
        s = lax.axis_index("s")
        w = s * _NC + c
        row0 = s * rows_per_tile
        tile0 = w * blk_per_w
        pltpu.sync_copy(ones_r, ones_v)
        pltpu.sync_copy(zeros_r.at[pl.ds(row0, rows_per_tile)],
                        accum.at[pl.ds(row0, rows_per_tile)])
        plsc.subcore_barrier()

        def fire_idx(k, buf):
            kc = jnp.minimum(k, nsup - 1)
            return pltpu.async_copy(src_r.at[pl.ds(tile0 + kc * kb, kb)],
                                    idx_v.at[pl.ds(buf * kb, kb)], isem)

        def wait_idx():
            pltpu.make_async_copy(src_r.at[pl.ds(tile0, kb)],
                                  idx_v.at[pl.ds(0, kb)], isem).wait()

        def wait_scatters():
            for _ in range(kb):
                pltpu.make_async_copy(ones_r, ones_v, ssem).wait()

        def round_body(g, first):
            b = lax.rem(g, 2)
            if not first:
                wait_scatters()
            wait_idx()
            fire_idx(g + 1, 1 - b)
            for j in range(kb):
                pltpu.async_copy(ones_v, accum.at[idx_v.at[b * kb + j]],
                                 ssem, add=True)

        fire_idx(0, 0)
        round_body(0, True)
        lax.fori_loop(1, nsup, lambda g, cr: (round_body(g, False), cr)[1], 0)
        wait_scatters()
        wait_idx()
        plsc.subcore_barrier()
        pltpu.sync_copy(accum.at[pl.ds(row0, rows_per_tile)],
                        out_r.at[pl.ds(c * _NPAD + row0, rows_per_tile)])

    fn = pl.kernel(
        body,
        out_type=jax.ShapeDtypeStruct((_NC * _NPAD, 16), jnp.float32),
        mesh=_mesh(),
        scratch_types=[
            pltpu.VMEM((2 * kb, _BLKW), jnp.int32),
            pltpu.VMEM((_BLKW, 16), jnp.float32),
            pltpu.VMEM_SHARED((_NPAD, 16), jnp.float32),
            pltpu.SemaphoreType.DMA,
            pltpu.SemaphoreType.DMA,
        ],
        compiler_params=pltpu.CompilerParams(use_tc_tiling_on_sc=False))
    return fn(src2d, ones, zeros)


def _mm(A, W, b=None, C=None, relu=False, C2=None, b2=None):
    n, k = A.shape
    m = W.shape[1]
    bn = 400
    grid = (n // bn,)
    in_specs = [pl.BlockSpec((bn, k), lambda i: (i, 0)),
                pl.BlockSpec((k, m), lambda i: (0, 0))]
    args = [A, W]
    if b is not None:
        in_specs.append(pl.BlockSpec((1, m), lambda i: (0, 0)))
        args.append(b.reshape(1, m))
    if C is not None:
        in_specs.append(pl.BlockSpec((bn, m), lambda i: (i, 0)))
        args.append(C)
    if C2 is not None:
        in_specs.append(pl.BlockSpec((bn, m), lambda i: (i, 0)))
        args.append(C2)
    if b2 is not None:
        in_specs.append(pl.BlockSpec((1, m), lambda i: (0, 0)))
        args.append(b2.reshape(1, m))

    def body(*refs):
        i = 2
        acc = jnp.dot(refs[0][...], refs[1][...],
                      preferred_element_type=jnp.float32)
        if b is not None:
            acc = acc + refs[i][...]
            i += 1
        if C is not None:
            acc = acc + refs[i][...]
            i += 1
        if relu:
            acc = jnp.maximum(acc, 0.0)
        if C2 is not None:
            acc = acc + refs[i][...]
            i += 1
        if b2 is not None:
            acc = acc + refs[i][...]
            i += 1
        refs[i][...] = acc

    return pl.pallas_call(
        body, grid=grid, in_specs=in_specs,
        out_specs=pl.BlockSpec((bn, m), lambda i: (i, 0)),
        out_shape=jax.ShapeDtypeStruct((n, m), jnp.float32))(*args)


def _table(u, n_chunks):
    fp = n_chunks * _FC
    t = jnp.pad(u, ((0, _NPAD - _N), (0, fp - u.shape[1])))
    return t.reshape(_NPAD, n_chunks, _FC).transpose(1, 0, 2).reshape(
        n_chunks * _NPAD, _FC)


def _untable(o, n_chunks, f):
    return o.reshape(n_chunks, _NPAD, _FC).transpose(1, 0, 2).reshape(
        _NPAD, n_chunks * _FC)[:_N, :f]


def kernel(x, edge_index, Wc1, bc1, Wc2, bc2, Wc3, bc3, W1, b1, W2, b2):
    f32 = jnp.float32
    src = edge_index[0].astype(jnp.int32)
    dst = edge_index[1].astype(jnp.int32)
    padi = jnp.full((_EPAD - _E,), _N, jnp.int32)
    src2d = jnp.concatenate([src, padi]).reshape(_NBLK, _BLKW)
    dst2d = jnp.concatenate([dst, padi]).reshape(_NBLK, _BLKW)
    zerosfc = jnp.zeros((_NPAD, _FC), f32)
    zeros16 = jnp.zeros((_NPAD, 16), f32)
    ones16 = jnp.ones((_BLKW, 16), f32)

    degp = _deg_call(src2d, ones16, zeros16)
    deg = degp[:_N, 0] + degp[_NPAD:_NPAD + _N, 0]
    dinv = jnp.where(deg > 0, 1.0 / jnp.sqrt(jnp.maximum(deg, 1e-12)), 0.0)
    d = dinv[:, None]
    d2 = (dinv * dinv)[:, None]

    def S(u, n_chunks, f):
        t = _table(u, n_chunks)
        o = _spmm_call(n_chunks, t, src2d, dst2d, zerosfc)
        return _untable(o, n_chunks, f)

    S1 = S(d * x, 4, 64)
    S2 = S(-d2 * S1, 4, 64)
    Wbig = jnp.concatenate([Wc1[0] - Wc1[2], W1, W2], axis=1)
    X700 = _mm(x, Wbig)
    Wm2 = jnp.concatenate([-Wc1[1], -2.0 * Wc1[2]], axis=0)
    A12 = jnp.concatenate([d * S1, d * S2], axis=1)
    out1 = _mm(A12, Wm2, b=bc1, C=X700[:, :300], relu=True,
               C2=X700[:, 300:600], b2=b1)

    M3 = _mm(out1, jnp.concatenate([Wc2[0], Wc2[1], Wc2[2]], axis=1))
    A0 = M3[:, :100]
    A1 = M3[:, 100:200]
    A2 = M3[:, 200:300]
    SB = S(d * A2, 8, 100)
    SCr = S(d * A1 - 2.0 * d2 * SB, 8, 100)
    out3 = (jnp.maximum(A0 - A2 - d * SCr + bc2, 0.0)
            + jnp.maximum(X700[:, 600:700] + b2, 0.0))

    M4 = _mm(out3, jnp.concatenate([Wc3[0], Wc3[1], Wc3[2]], axis=1))
    B0 = M4[:, :64]
    B1 = M4[:, 64:128]
    B2 = M4[:, 128:192]
    SD = S(d * B2, 4, 64)
    SE = S(d * B1 - 2.0 * d2 * SD, 4, 64)
    return B0 - B2 - d * SE + bc3
